# baseline (speedup 1.0000x reference)
import functools

import jax
import jax.numpy as jnp
from jax import lax
from jax.experimental import pallas as pl
from jax.experimental.pallas import tpu as pltpu
from jax.experimental.pallas import tpu_sc as plsc

EMBED_D = 64
GROUP = 128


@functools.lru_cache(maxsize=None)
def _emb_gather(num_idx: int):
    info = plsc.get_sparse_core_info()
    nc, ns = info.num_cores, info.num_subcores
    nw = nc * ns
    rows_total = num_idx // GROUP
    rpw = rows_total // nw
    bpw = num_idx // nw

    mesh = plsc.VectorSubcoreMesh(core_axis_name="c", subcore_axis_name="s")

    @functools.partial(
        pl.kernel,
        mesh=mesh,
        out_type=jax.ShapeDtypeStruct((num_idx, EMBED_D), jnp.float32),
        scratch_types=[
            pltpu.VMEM((rpw, GROUP), jnp.int32),
            pltpu.VMEM((GROUP, EMBED_D), jnp.float32),
            pltpu.SemaphoreType.DMA,
        ],
        compiler_params=pltpu.CompilerParams(use_tc_tiling_on_sc=False),
    )
    def k(idx_hbm, table_hbm, out_hbm, idx_v, rows_v, sem):
        wid = lax.axis_index("s") * nc + lax.axis_index("c")
        row0 = wid * rpw
        base = wid * bpw
        pltpu.sync_copy(idx_hbm.at[pl.ds(row0, rpw)], idx_v)

        def body(j, carry):
            pltpu.async_copy(table_hbm.at[idx_v.at[j]], rows_v, sem).wait()
            pltpu.sync_copy(rows_v, out_hbm.at[pl.ds(base + j * GROUP, GROUP)])
            return carry

        lax.fori_loop(0, rpw, body, 0)

    return k


def kernel(inputs, token_embed_weights):
    idx = inputs.astype(jnp.int32)
    num_idx = idx.size
    idx2 = idx.reshape(num_idx // GROUP, GROUP)
    out = _emb_gather(num_idx)(idx2, token_embed_weights)
    return out.reshape(inputs.shape + (EMBED_D,)), token_embed_weights


# speedup vs baseline: 1.0993x; 1.0993x over previous
import functools

import jax
import jax.numpy as jnp
from jax import lax
from jax.experimental import pallas as pl
from jax.experimental.pallas import tpu as pltpu
from jax.experimental.pallas import tpu_sc as plsc

EMBED_D = 64
GROUP = 128
GPC = 4
CHUNK = GROUP * GPC


@functools.lru_cache(maxsize=None)
def _emb_gather(num_idx: int):
    info = plsc.get_sparse_core_info()
    nc, ns = info.num_cores, info.num_subcores
    nw = nc * ns
    rows_total = num_idx // GROUP
    rpw = rows_total // nw
    bpw = num_idx // nw
    nchunks = bpw // CHUNK

    mesh = plsc.VectorSubcoreMesh(core_axis_name="c", subcore_axis_name="s")

    @functools.partial(
        pl.kernel,
        mesh=mesh,
        out_type=jax.ShapeDtypeStruct((num_idx, EMBED_D), jnp.float32),
        scratch_types=[
            pltpu.VMEM((rpw, GROUP), jnp.int32),
            pltpu.VMEM((2, CHUNK, EMBED_D), jnp.float32),
            pltpu.SemaphoreType.DMA((2,)),
            pltpu.SemaphoreType.DMA((2,)),
        ],
        compiler_params=pltpu.CompilerParams(use_tc_tiling_on_sc=False),
    )
    def k(idx_hbm, table_hbm, out_hbm, idx_v, rows_v, gsem, wsem):
        wid = lax.axis_index("s") * nc + lax.axis_index("c")
        row0 = wid * rpw
        base = wid * bpw
        pltpu.sync_copy(idx_hbm.at[pl.ds(row0, rpw)], idx_v)

        def fire_gathers(g, b):
            for j in range(GPC):
                pltpu.async_copy(
                    table_hbm.at[idx_v.at[g * GPC + j]],
                    rows_v.at[b, pl.ds(j * GROUP, GROUP)],
                    gsem.at[b],
                )

        def drain_gathers(g, b):
            for j in range(GPC):
                pltpu.make_async_copy(
                    table_hbm.at[idx_v.at[g * GPC + j]],
                    rows_v.at[b, pl.ds(j * GROUP, GROUP)],
                    gsem.at[b],
                ).wait()

        fire_gathers(0, 0)

        def body(g, carry):
            b = lax.rem(g, 2)
            nb = 1 - b
            drain_gathers(g, b)

            @pl.when(g >= 1)
            def _():
                pltpu.make_async_copy(
                    rows_v.at[nb],
                    out_hbm.at[pl.ds(base + (g - 1) * CHUNK, CHUNK)],
                    wsem.at[nb],
                ).wait()

            @pl.when(g + 1 < nchunks)
            def _():
                fire_gathers(g + 1, nb)

            pltpu.async_copy(
                rows_v.at[b],
                out_hbm.at[pl.ds(base + g * CHUNK, CHUNK)],
                wsem.at[b],
            )
            return carry

        lax.fori_loop(0, nchunks, body, 0)
        lastb = (nchunks - 1) % 2
        pltpu.make_async_copy(
            rows_v.at[lastb],
            out_hbm.at[pl.ds(base + (nchunks - 1) * CHUNK, CHUNK)],
            wsem.at[lastb],
        ).wait()

    return k


def kernel(inputs, token_embed_weights):
    idx = inputs.astype(jnp.int32)
    num_idx = idx.size
    idx2 = idx.reshape(num_idx // GROUP, GROUP)
    out = _emb_gather(num_idx)(idx2, token_embed_weights)
    return out.reshape(inputs.shape + (EMBED_D,)), token_embed_weights
